# baseline (speedup 1.0000x reference)
import functools

import jax
import jax.numpy as jnp
from jax import lax
from jax.experimental import pallas as pl
from jax.experimental.pallas import tpu as pltpu
from jax.experimental.pallas import tpu_sc as plsc

N = 10000
E = 640000
HID = 128
OUT = 256
LAYERS = 4
GRAPHS = 64

NW = 32
L = 128
CH = 160
E_PAD = NW * L * CH
N_PAD = 10112
RT = N_PAD // 16
IB = 16
CH0 = 80
CH1 = 240
DEGW = 128

_mesh = plsc.VectorSubcoreMesh(core_axis_name="c", subcore_axis_name="s")


@functools.partial(
    pl.kernel,
    mesh=_mesh,
    out_type=jax.ShapeDtypeStruct((2, N_PAD, HID), jnp.float32),
    scratch_types=[
        pltpu.VMEM((IB, L), jnp.int32),
        pltpu.VMEM((IB, L), jnp.int32),
        pltpu.VMEM((L, HID), jnp.float32),
        pltpu.VMEM((L, HID), jnp.float32),
        pltpu.VMEM_SHARED((N_PAD, HID), jnp.float32),
        pltpu.SemaphoreType.DMA,
        pltpu.SemaphoreType.DMA,
    ],
)
def _edge_scatter(g_h, src_h, dst_h, zeros_h, out_h,
                  src_v, dst_v, buf_a, buf_b, acc_sh, sem_a, sem_b):
    c = lax.axis_index("c")
    s = lax.axis_index("s")
    pltpu.sync_copy(zeros_h.at[pl.ds(s * RT, RT)], acc_sh.at[pl.ds(s * RT, RT)])
    plsc.subcore_barrier()
    nch = jnp.where(c == 0, CH0, CH1)
    base = jnp.where(c == 0, s * CH0, 16 * CH0 + s * CH1)

    def group(b, carry):
        pltpu.sync_copy(src_h.at[pl.ds(base + b * IB, IB)], src_v)
        pltpu.sync_copy(dst_h.at[pl.ds(base + b * IB, IB)], dst_v)
        cp0 = pltpu.async_copy(g_h.at[src_v.at[0]], buf_a, sem_a)

        def pair(q, carry2):
            pltpu.make_async_copy(g_h.at[src_v.at[2 * q]], buf_a, sem_a).wait()
            pltpu.async_copy(g_h.at[src_v.at[2 * q + 1]], buf_b, sem_b)
            pltpu.sync_copy(buf_a, acc_sh.at[dst_v.at[2 * q]], add=True)
            pltpu.make_async_copy(g_h.at[src_v.at[2 * q + 1]], buf_b, sem_b).wait()

            @pl.when(q < IB // 2 - 1)
            def _fire_next():
                pltpu.async_copy(g_h.at[src_v.at[2 * q + 2]], buf_a, sem_a)

            pltpu.sync_copy(buf_b, acc_sh.at[dst_v.at[2 * q + 1]], add=True)
            return carry2

        return lax.fori_loop(0, IB // 2, pair, carry)

    lax.fori_loop(0, nch // IB, group, 0)
    plsc.subcore_barrier()
    pltpu.sync_copy(acc_sh.at[pl.ds(s * RT, RT)],
                    out_h.at[c, pl.ds(s * RT, RT)])


@functools.partial(
    pl.kernel,
    mesh=_mesh,
    out_type=jax.ShapeDtypeStruct((2, N_PAD, DEGW), jnp.float32),
    scratch_types=[
        pltpu.VMEM((CH, L), jnp.int32),
        pltpu.VMEM((L, DEGW), jnp.float32),
        pltpu.VMEM_SHARED((N_PAD, DEGW), jnp.float32),
    ],
)
def _degree(dst_h, zeros_h, ones_h, out_h, dst_v, ones_v, acc_sh):
    c = lax.axis_index("c")
    s = lax.axis_index("s")
    wid = s * 2 + c
    pltpu.sync_copy(zeros_h.at[pl.ds(s * RT, RT)], acc_sh.at[pl.ds(s * RT, RT)])
    pltpu.sync_copy(dst_h.at[pl.ds(wid * CH, CH)], dst_v)
    pltpu.sync_copy(ones_h, ones_v)
    plsc.subcore_barrier()

    def chunk(j, carry):
        pltpu.sync_copy(ones_v, acc_sh.at[dst_v.at[j]], add=True)
        return carry

    lax.fori_loop(0, CH, chunk, 0)
    plsc.subcore_barrier()
    pltpu.sync_copy(acc_sh.at[pl.ds(s * RT, RT)],
                    out_h.at[c, pl.ds(s * RT, RT)])


_BS = 2000


def _pre_body(x_ref, w_ref, b_ref, o_ref):
    o_ref[...] = (jnp.dot(x_ref[...], w_ref[...],
                          preferred_element_type=jnp.float32) + b_ref[...])


def _prep_body(h_ref, w_ref, d0_ref, d1_ref, o_ref):
    dinv = lax.rsqrt(d0_ref[...] + d1_ref[...] + 1.0)
    t = jnp.dot(h_ref[...], w_ref[...], preferred_element_type=jnp.float32)
    o_ref[...] = t * dinv


def _post_body(a0_ref, a1_ref, g_ref, d0_ref, d1_ref, s_ref, b_ref, o_ref):
    dinv = lax.rsqrt(d0_ref[...] + d1_ref[...] + 1.0)
    conv = (a0_ref[...] + a1_ref[...] + g_ref[...]) * dinv
    o_ref[...] = jnp.maximum(conv * s_ref[...] + b_ref[...], 0.0)


def _pool_body(h_ref, brow_ref, bcol_ref, sum_ref, max_ref, cnt_ref):
    i = pl.program_id(0)

    @pl.when(i == 0)
    def _init():
        sum_ref[...] = jnp.zeros_like(sum_ref)
        cnt_ref[...] = jnp.zeros_like(cnt_ref)
        max_ref[...] = jnp.full_like(max_ref, -1e30)

    h = h_ref[...]
    ids_row = brow_ref[...].reshape(1, _BS)
    gid = lax.broadcasted_iota(jnp.int32, (GRAPHS, _BS), 0)
    onehot = (gid == ids_row).astype(jnp.float32)
    sum_ref[...] += jnp.dot(onehot, h, preferred_element_type=jnp.float32)
    cnt_ref[...] += jnp.sum(onehot, axis=1, keepdims=True)
    ids_col = bcol_ref[...]
    m = max_ref[...]
    rowid = lax.broadcasted_iota(jnp.int32, (GRAPHS, 1), 0)
    for g in range(GRAPHS):
        vals = jnp.where(ids_col == g, h, -1e30)
        mg = jnp.max(vals, axis=0, keepdims=True)
        m = jnp.where(rowid == g, jnp.maximum(m, mg), m)
    max_ref[...] = m


def _mlp_body(sum_ref, max_ref, cnt_ref, w1a_ref, w1b_ref, b1_ref,
              w2_ref, b2_ref, o_ref):
    gmean = sum_ref[...] / jnp.maximum(cnt_ref[...], 1.0)
    hmid = (jnp.dot(gmean, w1a_ref[...], preferred_element_type=jnp.float32)
            + jnp.dot(max_ref[...], w1b_ref[...],
                      preferred_element_type=jnp.float32)
            + b1_ref[...])
    hmid = jnp.maximum(hmid, 0.0)
    o_ref[...] = (jnp.dot(hmid, w2_ref[...],
                          preferred_element_type=jnp.float32) + b2_ref[...])


def _node_spec(i):
    return (i, 0)


def _fixed_spec(i):
    return (0, 0)


def kernel(x, edge_index, batch, Wp, bp, convW, convb, bn_gamma, bn_beta,
           bn_rm, bn_rv, Wa1, ba1, Wa2, ba2, Wo1, bo1, Wo2, bo2):
    f32 = jnp.float32
    src = edge_index[0]
    dst = edge_index[1]
    pad = E_PAD - E
    src2d = jnp.concatenate([src, jnp.zeros((pad,), jnp.int32)]).reshape(E_PAD // L, L)
    dst2d = jnp.concatenate([dst, jnp.full((pad,), N, jnp.int32)]).reshape(E_PAD // L, L)
    zeros_big = jnp.zeros((N_PAD, HID), f32)
    ones_deg = jnp.ones((L, DEGW), f32)

    degp = _degree(dst2d, zeros_big, ones_deg)
    d0 = degp[0, :N, :1]
    d1 = degp[1, :N, :1]

    x_p = jnp.pad(x, ((0, 0), (0, HID - x.shape[1])))
    Wp_p = jnp.pad(Wp, ((0, HID - Wp.shape[0]), (0, 0)))
    h = pl.pallas_call(
        _pre_body,
        grid=(N // _BS,),
        in_specs=[pl.BlockSpec((_BS, HID), _node_spec),
                  pl.BlockSpec((HID, HID), _fixed_spec),
                  pl.BlockSpec((1, HID), _fixed_spec)],
        out_specs=pl.BlockSpec((_BS, HID), _node_spec),
        out_shape=jax.ShapeDtypeStruct((N, HID), f32),
    )(x_p, Wp_p, bp[None, :])

    A = bn_gamma / jnp.sqrt(bn_rv + 1e-5)
    B = (convb - bn_rm) * A + bn_beta

    prep = pl.pallas_call(
        _prep_body,
        grid=(N // _BS,),
        in_specs=[pl.BlockSpec((_BS, HID), _node_spec),
                  pl.BlockSpec((HID, HID), _fixed_spec),
                  pl.BlockSpec((_BS, 1), _node_spec),
                  pl.BlockSpec((_BS, 1), _node_spec)],
        out_specs=pl.BlockSpec((_BS, HID), _node_spec),
        out_shape=jax.ShapeDtypeStruct((N, HID), f32),
    )

    post = pl.pallas_call(
        _post_body,
        grid=(N // _BS,),
        in_specs=[pl.BlockSpec((_BS, HID), _node_spec),
                  pl.BlockSpec((_BS, HID), _node_spec),
                  pl.BlockSpec((_BS, HID), _node_spec),
                  pl.BlockSpec((_BS, 1), _node_spec),
                  pl.BlockSpec((_BS, 1), _node_spec),
                  pl.BlockSpec((1, HID), _fixed_spec),
                  pl.BlockSpec((1, HID), _fixed_spec)],
        out_specs=pl.BlockSpec((_BS, HID), _node_spec),
        out_shape=jax.ShapeDtypeStruct((N, HID), f32),
    )

    for i in range(LAYERS):
        g = prep(h, convW[i], d0, d1)
        accp = _edge_scatter(g, src2d, dst2d, zeros_big)
        h = post(accp[0, :N], accp[1, :N], g, d0, d1,
                 A[i][None, :], B[i][None, :])

    node_features = h

    gsum, gmax, gcnt = pl.pallas_call(
        _pool_body,
        grid=(N // _BS,),
        in_specs=[pl.BlockSpec((_BS, HID), _node_spec),
                  pl.BlockSpec((1, 1, _BS), lambda i: (i, 0, 0)),
                  pl.BlockSpec((_BS, 1), _node_spec)],
        out_specs=[pl.BlockSpec((GRAPHS, HID), _fixed_spec),
                   pl.BlockSpec((GRAPHS, HID), _fixed_spec),
                   pl.BlockSpec((GRAPHS, 1), _fixed_spec)],
        out_shape=[jax.ShapeDtypeStruct((GRAPHS, HID), f32),
                   jax.ShapeDtypeStruct((GRAPHS, HID), f32),
                   jax.ShapeDtypeStruct((GRAPHS, 1), f32)],
    )(h, batch.reshape(N // _BS, 1, _BS), batch[:, None])

    mol = pl.pallas_call(
        _mlp_body,
        in_specs=[pl.BlockSpec((GRAPHS, HID), None),
                  pl.BlockSpec((GRAPHS, HID), None),
                  pl.BlockSpec((GRAPHS, 1), None),
                  pl.BlockSpec((HID, HID), None),
                  pl.BlockSpec((HID, HID), None),
                  pl.BlockSpec((1, HID), None),
                  pl.BlockSpec((HID, OUT), None),
                  pl.BlockSpec((1, OUT), None)],
        out_specs=pl.BlockSpec((GRAPHS, OUT), None),
        out_shape=jax.ShapeDtypeStruct((GRAPHS, OUT), f32),
    )(gsum, gmax, gcnt, Wo1[:HID], Wo1[HID:], bo1[None, :],
      Wo2, bo2[None, :])

    return (mol, node_features)


# speedup vs baseline: 1.0649x; 1.0649x over previous
import functools

import jax
import jax.numpy as jnp
from jax import lax
from jax.experimental import pallas as pl
from jax.experimental.pallas import tpu as pltpu
from jax.experimental.pallas import tpu_sc as plsc

N = 10000
E = 640000
HID = 128
OUT = 256
LAYERS = 4
GRAPHS = 64

NW = 32
L = 128
CH = 160
E_PAD = NW * L * CH
N_PAD = 10112
RT = N_PAD // 16
IB = 32
CH0 = 128
CH1 = 192
DEGW = 128

_mesh = plsc.VectorSubcoreMesh(core_axis_name="c", subcore_axis_name="s")


@functools.partial(
    pl.kernel,
    mesh=_mesh,
    out_type=jax.ShapeDtypeStruct((2, N_PAD, HID), jnp.float32),
    scratch_types=[
        pltpu.VMEM((IB, L), jnp.int32),
        pltpu.VMEM((IB, L), jnp.int32),
        pltpu.VMEM((L, HID), jnp.float32),
        pltpu.VMEM((L, HID), jnp.float32),
        pltpu.VMEM_SHARED((N_PAD, HID), jnp.float32),
        pltpu.SemaphoreType.DMA,
        pltpu.SemaphoreType.DMA,
    ],
)
def _edge_scatter(g_h, src_h, dst_h, zeros_h, out_h,
                  src_v, dst_v, buf_a, buf_b, acc_sh, sem_a, sem_b):
    c = lax.axis_index("c")
    s = lax.axis_index("s")
    pltpu.sync_copy(zeros_h.at[pl.ds(s * RT, RT)], acc_sh.at[pl.ds(s * RT, RT)])
    plsc.subcore_barrier()
    nch = jnp.where(c == 0, CH0, CH1)
    base = jnp.where(c == 0, s * CH0, 16 * CH0 + s * CH1)

    def group(b, carry):
        pltpu.sync_copy(src_h.at[pl.ds(base + b * IB, IB)], src_v)
        pltpu.sync_copy(dst_h.at[pl.ds(base + b * IB, IB)], dst_v)
        cp0 = pltpu.async_copy(g_h.at[src_v.at[0]], buf_a, sem_a)

        def pair(q, carry2):
            pltpu.make_async_copy(g_h.at[src_v.at[2 * q]], buf_a, sem_a).wait()
            pltpu.async_copy(g_h.at[src_v.at[2 * q + 1]], buf_b, sem_b)
            pltpu.sync_copy(buf_a, acc_sh.at[dst_v.at[2 * q]], add=True)
            pltpu.make_async_copy(g_h.at[src_v.at[2 * q + 1]], buf_b, sem_b).wait()

            @pl.when(q < IB // 2 - 1)
            def _fire_next():
                pltpu.async_copy(g_h.at[src_v.at[2 * q + 2]], buf_a, sem_a)

            pltpu.sync_copy(buf_b, acc_sh.at[dst_v.at[2 * q + 1]], add=True)
            return carry2

        return lax.fori_loop(0, IB // 2, pair, carry)

    lax.fori_loop(0, nch // IB, group, 0)
    plsc.subcore_barrier()
    pltpu.sync_copy(acc_sh.at[pl.ds(s * RT, RT)],
                    out_h.at[c, pl.ds(s * RT, RT)])


@functools.partial(
    pl.kernel,
    mesh=_mesh,
    out_type=jax.ShapeDtypeStruct((2, N_PAD, DEGW), jnp.float32),
    scratch_types=[
        pltpu.VMEM((CH, L), jnp.int32),
        pltpu.VMEM((L, DEGW), jnp.float32),
        pltpu.VMEM_SHARED((N_PAD, DEGW), jnp.float32),
    ],
)
def _degree(dst_h, zeros_h, ones_h, out_h, dst_v, ones_v, acc_sh):
    c = lax.axis_index("c")
    s = lax.axis_index("s")
    wid = s * 2 + c
    pltpu.sync_copy(zeros_h.at[pl.ds(s * RT, RT)], acc_sh.at[pl.ds(s * RT, RT)])
    pltpu.sync_copy(dst_h.at[pl.ds(wid * CH, CH)], dst_v)
    pltpu.sync_copy(ones_h, ones_v)
    plsc.subcore_barrier()

    def chunk(j, carry):
        pltpu.sync_copy(ones_v, acc_sh.at[dst_v.at[j]], add=True)
        return carry

    lax.fori_loop(0, CH, chunk, 0)
    plsc.subcore_barrier()
    pltpu.sync_copy(acc_sh.at[pl.ds(s * RT, RT)],
                    out_h.at[c, pl.ds(s * RT, RT)])


_BS = 2000


def _pre_body(x_ref, w_ref, b_ref, o_ref):
    o_ref[...] = (jnp.dot(x_ref[...], w_ref[...],
                          preferred_element_type=jnp.float32) + b_ref[...])


def _prep_body(h_ref, w_ref, d0_ref, d1_ref, o_ref):
    dinv = lax.rsqrt(d0_ref[...] + d1_ref[...] + 1.0)
    t = jnp.dot(h_ref[...], w_ref[...], preferred_element_type=jnp.float32)
    o_ref[...] = t * dinv


def _post_body(a0_ref, a1_ref, g_ref, d0_ref, d1_ref, s_ref, b_ref, o_ref):
    dinv = lax.rsqrt(d0_ref[...] + d1_ref[...] + 1.0)
    conv = (a0_ref[...] + a1_ref[...] + g_ref[...]) * dinv
    o_ref[...] = jnp.maximum(conv * s_ref[...] + b_ref[...], 0.0)


def _pool_body(h_ref, brow_ref, bcol_ref, sum_ref, max_ref, cnt_ref):
    i = pl.program_id(0)

    @pl.when(i == 0)
    def _init():
        sum_ref[...] = jnp.zeros_like(sum_ref)
        cnt_ref[...] = jnp.zeros_like(cnt_ref)
        max_ref[...] = jnp.full_like(max_ref, -1e30)

    h = h_ref[...]
    ids_row = brow_ref[...].reshape(1, _BS)
    gid = lax.broadcasted_iota(jnp.int32, (GRAPHS, _BS), 0)
    onehot = (gid == ids_row).astype(jnp.float32)
    sum_ref[...] += jnp.dot(onehot, h, preferred_element_type=jnp.float32)
    cnt_ref[...] += jnp.sum(onehot, axis=1, keepdims=True)
    ids_col = bcol_ref[...]
    m = max_ref[...]
    rowid = lax.broadcasted_iota(jnp.int32, (GRAPHS, 1), 0)
    for g in range(GRAPHS):
        vals = jnp.where(ids_col == g, h, -1e30)
        mg = jnp.max(vals, axis=0, keepdims=True)
        m = jnp.where(rowid == g, jnp.maximum(m, mg), m)
    max_ref[...] = m


def _mlp_body(sum_ref, max_ref, cnt_ref, w1a_ref, w1b_ref, b1_ref,
              w2_ref, b2_ref, o_ref):
    gmean = sum_ref[...] / jnp.maximum(cnt_ref[...], 1.0)
    hmid = (jnp.dot(gmean, w1a_ref[...], preferred_element_type=jnp.float32)
            + jnp.dot(max_ref[...], w1b_ref[...],
                      preferred_element_type=jnp.float32)
            + b1_ref[...])
    hmid = jnp.maximum(hmid, 0.0)
    o_ref[...] = (jnp.dot(hmid, w2_ref[...],
                          preferred_element_type=jnp.float32) + b2_ref[...])


def _node_spec(i):
    return (i, 0)


def _fixed_spec(i):
    return (0, 0)


def kernel(x, edge_index, batch, Wp, bp, convW, convb, bn_gamma, bn_beta,
           bn_rm, bn_rv, Wa1, ba1, Wa2, ba2, Wo1, bo1, Wo2, bo2):
    f32 = jnp.float32
    src = edge_index[0]
    dst = edge_index[1]
    pad = E_PAD - E
    src2d = jnp.concatenate([src, jnp.zeros((pad,), jnp.int32)]).reshape(E_PAD // L, L)
    dst2d = jnp.concatenate([dst, jnp.full((pad,), N, jnp.int32)]).reshape(E_PAD // L, L)
    zeros_big = jnp.zeros((N_PAD, HID), f32)
    ones_deg = jnp.ones((L, DEGW), f32)

    degp = _degree(dst2d, zeros_big, ones_deg)
    d0 = degp[0, :N, :1]
    d1 = degp[1, :N, :1]

    x_p = jnp.pad(x, ((0, 0), (0, HID - x.shape[1])))
    Wp_p = jnp.pad(Wp, ((0, HID - Wp.shape[0]), (0, 0)))
    h = pl.pallas_call(
        _pre_body,
        grid=(N // _BS,),
        in_specs=[pl.BlockSpec((_BS, HID), _node_spec),
                  pl.BlockSpec((HID, HID), _fixed_spec),
                  pl.BlockSpec((1, HID), _fixed_spec)],
        out_specs=pl.BlockSpec((_BS, HID), _node_spec),
        out_shape=jax.ShapeDtypeStruct((N, HID), f32),
    )(x_p, Wp_p, bp[None, :])

    A = bn_gamma / jnp.sqrt(bn_rv + 1e-5)
    B = (convb - bn_rm) * A + bn_beta

    prep = pl.pallas_call(
        _prep_body,
        grid=(N // _BS,),
        in_specs=[pl.BlockSpec((_BS, HID), _node_spec),
                  pl.BlockSpec((HID, HID), _fixed_spec),
                  pl.BlockSpec((_BS, 1), _node_spec),
                  pl.BlockSpec((_BS, 1), _node_spec)],
        out_specs=pl.BlockSpec((_BS, HID), _node_spec),
        out_shape=jax.ShapeDtypeStruct((N, HID), f32),
    )

    post = pl.pallas_call(
        _post_body,
        grid=(N // _BS,),
        in_specs=[pl.BlockSpec((_BS, HID), _node_spec),
                  pl.BlockSpec((_BS, HID), _node_spec),
                  pl.BlockSpec((_BS, HID), _node_spec),
                  pl.BlockSpec((_BS, 1), _node_spec),
                  pl.BlockSpec((_BS, 1), _node_spec),
                  pl.BlockSpec((1, HID), _fixed_spec),
                  pl.BlockSpec((1, HID), _fixed_spec)],
        out_specs=pl.BlockSpec((_BS, HID), _node_spec),
        out_shape=jax.ShapeDtypeStruct((N, HID), f32),
    )

    for i in range(LAYERS):
        g = prep(h, convW[i], d0, d1)
        accp = _edge_scatter(g, src2d, dst2d, zeros_big)
        h = post(accp[0, :N], accp[1, :N], g, d0, d1,
                 A[i][None, :], B[i][None, :])

    node_features = h

    gsum, gmax, gcnt = pl.pallas_call(
        _pool_body,
        grid=(N // _BS,),
        in_specs=[pl.BlockSpec((_BS, HID), _node_spec),
                  pl.BlockSpec((1, 1, _BS), lambda i: (i, 0, 0)),
                  pl.BlockSpec((_BS, 1), _node_spec)],
        out_specs=[pl.BlockSpec((GRAPHS, HID), _fixed_spec),
                   pl.BlockSpec((GRAPHS, HID), _fixed_spec),
                   pl.BlockSpec((GRAPHS, 1), _fixed_spec)],
        out_shape=[jax.ShapeDtypeStruct((GRAPHS, HID), f32),
                   jax.ShapeDtypeStruct((GRAPHS, HID), f32),
                   jax.ShapeDtypeStruct((GRAPHS, 1), f32)],
    )(h, batch.reshape(N // _BS, 1, _BS), batch[:, None])

    mol = pl.pallas_call(
        _mlp_body,
        in_specs=[pl.BlockSpec((GRAPHS, HID), None),
                  pl.BlockSpec((GRAPHS, HID), None),
                  pl.BlockSpec((GRAPHS, 1), None),
                  pl.BlockSpec((HID, HID), None),
                  pl.BlockSpec((HID, HID), None),
                  pl.BlockSpec((1, HID), None),
                  pl.BlockSpec((HID, OUT), None),
                  pl.BlockSpec((1, OUT), None)],
        out_specs=pl.BlockSpec((GRAPHS, OUT), None),
        out_shape=jax.ShapeDtypeStruct((GRAPHS, OUT), f32),
    )(gsum, gmax, gcnt, Wo1[:HID], Wo1[HID:], bo1[None, :],
      Wo2, bo2[None, :])

    return (mol, node_features)


# speedup vs baseline: 1.0868x; 1.0206x over previous
import functools

import jax
import jax.numpy as jnp
from jax import lax
from jax.experimental import pallas as pl
from jax.experimental.pallas import tpu as pltpu
from jax.experimental.pallas import tpu_sc as plsc

N = 10000
E = 640000
HID = 128
OUT = 256
LAYERS = 4
GRAPHS = 64

NW = 32
L = 128
CH = 160
E_PAD = NW * L * CH
N_PAD = 10112
RT = N_PAD // 16
IB = 32
DEGW = 128

_mesh = plsc.VectorSubcoreMesh(core_axis_name="c", subcore_axis_name="s")


@functools.partial(
    pl.kernel,
    mesh=_mesh,
    out_type=jax.ShapeDtypeStruct((2, N_PAD, HID), jnp.float32),
    scratch_types=[
        pltpu.VMEM((IB, L), jnp.int32),
        pltpu.VMEM((IB, L), jnp.int32),
        pltpu.VMEM((L, HID), jnp.float32),
        pltpu.VMEM((L, HID), jnp.float32),
        pltpu.VMEM_SHARED((N_PAD, HID), jnp.float32),
        pltpu.SemaphoreType.DMA,
        pltpu.SemaphoreType.DMA,
    ],
)
def _edge_scatter(g_h, src_h, dst_h, zeros_h, out_h,
                  src_v, dst_v, buf_a, buf_b, acc_sh, sem_a, sem_b):
    c = lax.axis_index("c")
    s = lax.axis_index("s")
    wid = s * 2 + c
    pltpu.sync_copy(zeros_h.at[pl.ds(s * RT, RT)], acc_sh.at[pl.ds(s * RT, RT)])
    plsc.subcore_barrier()
    base = wid * CH

    def group(b, carry):
        pltpu.sync_copy(src_h.at[pl.ds(base + b * IB, IB)], src_v)
        pltpu.sync_copy(dst_h.at[pl.ds(base + b * IB, IB)], dst_v)
        cp0 = pltpu.async_copy(g_h.at[src_v.at[0]], buf_a, sem_a)

        def pair(q, carry2):
            pltpu.make_async_copy(g_h.at[src_v.at[2 * q]], buf_a, sem_a).wait()
            pltpu.async_copy(g_h.at[src_v.at[2 * q + 1]], buf_b, sem_b)
            pltpu.sync_copy(buf_a, acc_sh.at[dst_v.at[2 * q]], add=True)
            pltpu.make_async_copy(g_h.at[src_v.at[2 * q + 1]], buf_b, sem_b).wait()

            @pl.when(q < IB // 2 - 1)
            def _fire_next():
                pltpu.async_copy(g_h.at[src_v.at[2 * q + 2]], buf_a, sem_a)

            pltpu.sync_copy(buf_b, acc_sh.at[dst_v.at[2 * q + 1]], add=True)
            return carry2

        return lax.fori_loop(0, IB // 2, pair, carry)

    lax.fori_loop(0, CH // IB, group, 0)
    plsc.subcore_barrier()
    pltpu.sync_copy(acc_sh.at[pl.ds(s * RT, RT)],
                    out_h.at[c, pl.ds(s * RT, RT)])


@functools.partial(
    pl.kernel,
    mesh=_mesh,
    out_type=jax.ShapeDtypeStruct((2, N_PAD, DEGW), jnp.float32),
    scratch_types=[
        pltpu.VMEM((CH, L), jnp.int32),
        pltpu.VMEM((L, DEGW), jnp.float32),
        pltpu.VMEM_SHARED((N_PAD, DEGW), jnp.float32),
    ],
)
def _degree(dst_h, zeros_h, ones_h, out_h, dst_v, ones_v, acc_sh):
    c = lax.axis_index("c")
    s = lax.axis_index("s")
    wid = s * 2 + c
    pltpu.sync_copy(zeros_h.at[pl.ds(s * RT, RT)], acc_sh.at[pl.ds(s * RT, RT)])
    pltpu.sync_copy(dst_h.at[pl.ds(wid * CH, CH)], dst_v)
    pltpu.sync_copy(ones_h, ones_v)
    plsc.subcore_barrier()

    def chunk(j, carry):
        pltpu.sync_copy(ones_v, acc_sh.at[dst_v.at[j]], add=True)
        return carry

    lax.fori_loop(0, CH, chunk, 0)
    plsc.subcore_barrier()
    pltpu.sync_copy(acc_sh.at[pl.ds(s * RT, RT)],
                    out_h.at[c, pl.ds(s * RT, RT)])


_BS = 2000


def _preprep_body(x_ref, wp_ref, b_ref, w1_ref, d0_ref, d1_ref,
                  h_ref, g_ref):
    dinv = lax.rsqrt(d0_ref[...] + d1_ref[...] + 1.0)
    h = (jnp.dot(x_ref[...], wp_ref[...],
                 preferred_element_type=jnp.float32) + b_ref[...])
    h_ref[...] = h
    g_ref[...] = jnp.dot(h, w1_ref[...],
                         preferred_element_type=jnp.float32) * dinv


def _postprep_body(a0_ref, a1_ref, g_ref, d0_ref, d1_ref, s_ref, b_ref,
                   w_ref, h_ref, gn_ref):
    dinv = lax.rsqrt(d0_ref[...] + d1_ref[...] + 1.0)
    conv = (a0_ref[...] + a1_ref[...] + g_ref[...]) * dinv
    h = jnp.maximum(conv * s_ref[...] + b_ref[...], 0.0)
    h_ref[...] = h
    gn_ref[...] = jnp.dot(h, w_ref[...],
                          preferred_element_type=jnp.float32) * dinv


def _post_body(a0_ref, a1_ref, g_ref, d0_ref, d1_ref, s_ref, b_ref, o_ref):
    dinv = lax.rsqrt(d0_ref[...] + d1_ref[...] + 1.0)
    conv = (a0_ref[...] + a1_ref[...] + g_ref[...]) * dinv
    o_ref[...] = jnp.maximum(conv * s_ref[...] + b_ref[...], 0.0)


def _pool_body(h_ref, brow_ref, bcol_ref, sum_ref, max_ref, cnt_ref):
    i = pl.program_id(0)

    @pl.when(i == 0)
    def _init():
        sum_ref[...] = jnp.zeros_like(sum_ref)
        cnt_ref[...] = jnp.zeros_like(cnt_ref)
        max_ref[...] = jnp.full_like(max_ref, -1e30)

    h = h_ref[...]
    ids_row = brow_ref[...].reshape(1, _BS)
    gid = lax.broadcasted_iota(jnp.int32, (GRAPHS, _BS), 0)
    onehot = (gid == ids_row).astype(jnp.float32)
    sum_ref[...] += jnp.dot(onehot, h, preferred_element_type=jnp.float32)
    cnt_ref[...] += jnp.sum(onehot, axis=1, keepdims=True)
    ids_col = bcol_ref[...]
    m = max_ref[...]
    rowid = lax.broadcasted_iota(jnp.int32, (GRAPHS, 1), 0)
    for g in range(GRAPHS):
        vals = jnp.where(ids_col == g, h, -1e30)
        mg = jnp.max(vals, axis=0, keepdims=True)
        m = jnp.where(rowid == g, jnp.maximum(m, mg), m)
    max_ref[...] = m


def _mlp_body(sum_ref, max_ref, cnt_ref, w1a_ref, w1b_ref, b1_ref,
              w2_ref, b2_ref, o_ref):
    gmean = sum_ref[...] / jnp.maximum(cnt_ref[...], 1.0)
    hmid = (jnp.dot(gmean, w1a_ref[...], preferred_element_type=jnp.float32)
            + jnp.dot(max_ref[...], w1b_ref[...],
                      preferred_element_type=jnp.float32)
            + b1_ref[...])
    hmid = jnp.maximum(hmid, 0.0)
    o_ref[...] = (jnp.dot(hmid, w2_ref[...],
                          preferred_element_type=jnp.float32) + b2_ref[...])


def _node_spec(i):
    return (i, 0)


def _fixed_spec(i):
    return (0, 0)


def kernel(x, edge_index, batch, Wp, bp, convW, convb, bn_gamma, bn_beta,
           bn_rm, bn_rv, Wa1, ba1, Wa2, ba2, Wo1, bo1, Wo2, bo2):
    f32 = jnp.float32
    src = edge_index[0]
    dst = edge_index[1]
    pad = E_PAD - E
    src2d = jnp.concatenate([src, jnp.zeros((pad,), jnp.int32)]).reshape(E_PAD // L, L)
    dst2d = jnp.concatenate([dst, jnp.full((pad,), N, jnp.int32)]).reshape(E_PAD // L, L)
    zeros_big = jnp.zeros((N_PAD, HID), f32)
    ones_deg = jnp.ones((L, DEGW), f32)

    degp = _degree(dst2d, zeros_big, ones_deg)
    d0 = degp[0, :N, :1]
    d1 = degp[1, :N, :1]

    A = bn_gamma / jnp.sqrt(bn_rv + 1e-5)
    B = (convb - bn_rm) * A + bn_beta

    x_p = jnp.pad(x, ((0, 0), (0, HID - x.shape[1])))
    Wp_p = jnp.pad(Wp, ((0, HID - Wp.shape[0]), (0, 0)))
    h, g = pl.pallas_call(
        _preprep_body,
        grid=(N // _BS,),
        in_specs=[pl.BlockSpec((_BS, HID), _node_spec),
                  pl.BlockSpec((HID, HID), _fixed_spec),
                  pl.BlockSpec((1, HID), _fixed_spec),
                  pl.BlockSpec((HID, HID), _fixed_spec),
                  pl.BlockSpec((_BS, 1), _node_spec),
                  pl.BlockSpec((_BS, 1), _node_spec)],
        out_specs=[pl.BlockSpec((_BS, HID), _node_spec),
                   pl.BlockSpec((_BS, HID), _node_spec)],
        out_shape=[jax.ShapeDtypeStruct((N, HID), f32),
                   jax.ShapeDtypeStruct((N, HID), f32)],
    )(x_p, Wp_p, bp[None, :], convW[0], d0, d1)

    postprep = pl.pallas_call(
        _postprep_body,
        grid=(N // _BS,),
        in_specs=[pl.BlockSpec((_BS, HID), _node_spec),
                  pl.BlockSpec((_BS, HID), _node_spec),
                  pl.BlockSpec((_BS, HID), _node_spec),
                  pl.BlockSpec((_BS, 1), _node_spec),
                  pl.BlockSpec((_BS, 1), _node_spec),
                  pl.BlockSpec((1, HID), _fixed_spec),
                  pl.BlockSpec((1, HID), _fixed_spec),
                  pl.BlockSpec((HID, HID), _fixed_spec)],
        out_specs=[pl.BlockSpec((_BS, HID), _node_spec),
                   pl.BlockSpec((_BS, HID), _node_spec)],
        out_shape=[jax.ShapeDtypeStruct((N, HID), f32),
                   jax.ShapeDtypeStruct((N, HID), f32)],
    )

    post = pl.pallas_call(
        _post_body,
        grid=(N // _BS,),
        in_specs=[pl.BlockSpec((_BS, HID), _node_spec),
                  pl.BlockSpec((_BS, HID), _node_spec),
                  pl.BlockSpec((_BS, HID), _node_spec),
                  pl.BlockSpec((_BS, 1), _node_spec),
                  pl.BlockSpec((_BS, 1), _node_spec),
                  pl.BlockSpec((1, HID), _fixed_spec),
                  pl.BlockSpec((1, HID), _fixed_spec)],
        out_specs=pl.BlockSpec((_BS, HID), _node_spec),
        out_shape=jax.ShapeDtypeStruct((N, HID), f32),
    )

    for i in range(LAYERS):
        accp = _edge_scatter(g, src2d, dst2d, zeros_big)
        if i < LAYERS - 1:
            h, g = postprep(accp[0, :N], accp[1, :N], g, d0, d1,
                            A[i][None, :], B[i][None, :], convW[i + 1])
        else:
            h = post(accp[0, :N], accp[1, :N], g, d0, d1,
                     A[i][None, :], B[i][None, :])

    node_features = h

    gsum, gmax, gcnt = pl.pallas_call(
        _pool_body,
        grid=(N // _BS,),
        in_specs=[pl.BlockSpec((_BS, HID), _node_spec),
                  pl.BlockSpec((1, 1, _BS), lambda i: (i, 0, 0)),
                  pl.BlockSpec((_BS, 1), _node_spec)],
        out_specs=[pl.BlockSpec((GRAPHS, HID), _fixed_spec),
                   pl.BlockSpec((GRAPHS, HID), _fixed_spec),
                   pl.BlockSpec((GRAPHS, 1), _fixed_spec)],
        out_shape=[jax.ShapeDtypeStruct((GRAPHS, HID), f32),
                   jax.ShapeDtypeStruct((GRAPHS, HID), f32),
                   jax.ShapeDtypeStruct((GRAPHS, 1), f32)],
    )(h, batch.reshape(N // _BS, 1, _BS), batch[:, None])

    mol = pl.pallas_call(
        _mlp_body,
        in_specs=[pl.BlockSpec((GRAPHS, HID), None),
                  pl.BlockSpec((GRAPHS, HID), None),
                  pl.BlockSpec((GRAPHS, 1), None),
                  pl.BlockSpec((HID, HID), None),
                  pl.BlockSpec((HID, HID), None),
                  pl.BlockSpec((1, HID), None),
                  pl.BlockSpec((HID, OUT), None),
                  pl.BlockSpec((1, OUT), None)],
        out_specs=pl.BlockSpec((GRAPHS, OUT), None),
        out_shape=jax.ShapeDtypeStruct((GRAPHS, OUT), f32),
    )(gsum, gmax, gcnt, Wo1[:HID], Wo1[HID:], bo1[None, :],
      Wo2, bo2[None, :])

    return (mol, node_features)


# speedup vs baseline: 1.1223x; 1.0326x over previous
import functools

import jax
import jax.numpy as jnp
from jax import lax
from jax.experimental import pallas as pl
from jax.experimental.pallas import tpu as pltpu
from jax.experimental.pallas import tpu_sc as plsc

N = 10000
E = 640000
HID = 128
OUT = 256
LAYERS = 4
GRAPHS = 64

NW = 32
L = 128
CH = 160
E_PAD = NW * L * CH
N_PAD = 10112
RT = N_PAD // 16
IB = 32
DEGW = 128

_mesh = plsc.VectorSubcoreMesh(core_axis_name="c", subcore_axis_name="s")


@functools.partial(
    pl.kernel,
    mesh=_mesh,
    out_type=jax.ShapeDtypeStruct((2, N_PAD, HID), jnp.float32),
    scratch_types=[
        pltpu.VMEM((IB, L), jnp.int32),
        pltpu.VMEM((IB, L), jnp.int32),
        pltpu.VMEM((L, HID), jnp.float32),
        pltpu.VMEM((L, HID), jnp.float32),
        pltpu.VMEM_SHARED((N_PAD, HID), jnp.float32),
        pltpu.SemaphoreType.DMA,
        pltpu.SemaphoreType.DMA,
    ],
)
def _edge_scatter(g_h, src_h, dst_h, zeros_h, out_h,
                  src_v, dst_v, buf_a, buf_b, acc_sh, sem_a, sem_b):
    c = lax.axis_index("c")
    s = lax.axis_index("s")
    wid = s * 2 + c
    pltpu.sync_copy(zeros_h.at[pl.ds(s * RT, RT)], acc_sh.at[pl.ds(s * RT, RT)])
    plsc.subcore_barrier()
    base = wid * CH

    def group(b, carry):
        pltpu.sync_copy(src_h.at[pl.ds(base + b * IB, IB)], src_v)
        pltpu.sync_copy(dst_h.at[pl.ds(base + b * IB, IB)], dst_v)
        pltpu.async_copy(g_h.at[src_v.at[0]], buf_a, sem_a)
        pltpu.async_copy(g_h.at[src_v.at[1]], buf_b, sem_b)

        def pair(q, carry2):
            pltpu.make_async_copy(g_h.at[src_v.at[2 * q]], buf_a, sem_a).wait()
            pltpu.sync_copy(buf_a, acc_sh.at[dst_v.at[2 * q]], add=True)

            @pl.when(q < IB // 2 - 1)
            def _fire_a():
                pltpu.async_copy(g_h.at[src_v.at[2 * q + 2]], buf_a, sem_a)

            pltpu.make_async_copy(g_h.at[src_v.at[2 * q + 1]], buf_b, sem_b).wait()
            pltpu.sync_copy(buf_b, acc_sh.at[dst_v.at[2 * q + 1]], add=True)

            @pl.when(q < IB // 2 - 1)
            def _fire_b():
                pltpu.async_copy(g_h.at[src_v.at[2 * q + 3]], buf_b, sem_b)

            return carry2

        return lax.fori_loop(0, IB // 2, pair, carry)

    lax.fori_loop(0, CH // IB, group, 0)
    plsc.subcore_barrier()
    pltpu.sync_copy(acc_sh.at[pl.ds(s * RT, RT)],
                    out_h.at[c, pl.ds(s * RT, RT)])


@functools.partial(
    pl.kernel,
    mesh=_mesh,
    out_type=jax.ShapeDtypeStruct((2, N_PAD, DEGW), jnp.float32),
    scratch_types=[
        pltpu.VMEM((CH, L), jnp.int32),
        pltpu.VMEM((L, DEGW), jnp.float32),
        pltpu.VMEM_SHARED((N_PAD, DEGW), jnp.float32),
    ],
)
def _degree(dst_h, zeros_h, ones_h, out_h, dst_v, ones_v, acc_sh):
    c = lax.axis_index("c")
    s = lax.axis_index("s")
    wid = s * 2 + c
    pltpu.sync_copy(zeros_h.at[pl.ds(s * RT, RT)], acc_sh.at[pl.ds(s * RT, RT)])
    pltpu.sync_copy(dst_h.at[pl.ds(wid * CH, CH)], dst_v)
    pltpu.sync_copy(ones_h, ones_v)
    plsc.subcore_barrier()

    def chunk(j, carry):
        pltpu.sync_copy(ones_v, acc_sh.at[dst_v.at[j]], add=True)
        return carry

    lax.fori_loop(0, CH, chunk, 0)
    plsc.subcore_barrier()
    pltpu.sync_copy(acc_sh.at[pl.ds(s * RT, RT)],
                    out_h.at[c, pl.ds(s * RT, RT)])


_BS = 2000


def _preprep_body(x_ref, wp_ref, b_ref, w1_ref, d0_ref, d1_ref,
                  h_ref, g_ref):
    dinv = lax.rsqrt(d0_ref[...] + d1_ref[...] + 1.0)
    h = (jnp.dot(x_ref[...], wp_ref[...],
                 preferred_element_type=jnp.float32) + b_ref[...])
    h_ref[...] = h
    g_ref[...] = jnp.dot(h, w1_ref[...],
                         preferred_element_type=jnp.float32) * dinv


def _postprep_body(a0_ref, a1_ref, g_ref, d0_ref, d1_ref, s_ref, b_ref,
                   w_ref, h_ref, gn_ref):
    dinv = lax.rsqrt(d0_ref[...] + d1_ref[...] + 1.0)
    conv = (a0_ref[...] + a1_ref[...] + g_ref[...]) * dinv
    h = jnp.maximum(conv * s_ref[...] + b_ref[...], 0.0)
    h_ref[...] = h
    gn_ref[...] = jnp.dot(h, w_ref[...],
                          preferred_element_type=jnp.float32) * dinv


def _post_body(a0_ref, a1_ref, g_ref, d0_ref, d1_ref, s_ref, b_ref, o_ref):
    dinv = lax.rsqrt(d0_ref[...] + d1_ref[...] + 1.0)
    conv = (a0_ref[...] + a1_ref[...] + g_ref[...]) * dinv
    o_ref[...] = jnp.maximum(conv * s_ref[...] + b_ref[...], 0.0)


def _pool_body(h_ref, brow_ref, bcol_ref, sum_ref, max_ref, cnt_ref):
    i = pl.program_id(0)

    @pl.when(i == 0)
    def _init():
        sum_ref[...] = jnp.zeros_like(sum_ref)
        cnt_ref[...] = jnp.zeros_like(cnt_ref)
        max_ref[...] = jnp.full_like(max_ref, -1e30)

    h = h_ref[...]
    ids_row = brow_ref[...].reshape(1, _BS)
    gid = lax.broadcasted_iota(jnp.int32, (GRAPHS, _BS), 0)
    onehot = (gid == ids_row).astype(jnp.float32)
    sum_ref[...] += jnp.dot(onehot, h, preferred_element_type=jnp.float32)
    cnt_ref[...] += jnp.sum(onehot, axis=1, keepdims=True)
    ids_col = bcol_ref[...]
    m = max_ref[...]
    rowid = lax.broadcasted_iota(jnp.int32, (GRAPHS, 1), 0)
    for g in range(GRAPHS):
        vals = jnp.where(ids_col == g, h, -1e30)
        mg = jnp.max(vals, axis=0, keepdims=True)
        m = jnp.where(rowid == g, jnp.maximum(m, mg), m)
    max_ref[...] = m


def _mlp_body(sum_ref, max_ref, cnt_ref, w1a_ref, w1b_ref, b1_ref,
              w2_ref, b2_ref, o_ref):
    gmean = sum_ref[...] / jnp.maximum(cnt_ref[...], 1.0)
    hmid = (jnp.dot(gmean, w1a_ref[...], preferred_element_type=jnp.float32)
            + jnp.dot(max_ref[...], w1b_ref[...],
                      preferred_element_type=jnp.float32)
            + b1_ref[...])
    hmid = jnp.maximum(hmid, 0.0)
    o_ref[...] = (jnp.dot(hmid, w2_ref[...],
                          preferred_element_type=jnp.float32) + b2_ref[...])


def _node_spec(i):
    return (i, 0)


def _fixed_spec(i):
    return (0, 0)


def kernel(x, edge_index, batch, Wp, bp, convW, convb, bn_gamma, bn_beta,
           bn_rm, bn_rv, Wa1, ba1, Wa2, ba2, Wo1, bo1, Wo2, bo2):
    f32 = jnp.float32
    src = edge_index[0]
    dst = edge_index[1]
    pad = E_PAD - E
    src2d = jnp.concatenate([src, jnp.zeros((pad,), jnp.int32)]).reshape(E_PAD // L, L)
    dst2d = jnp.concatenate([dst, jnp.full((pad,), N, jnp.int32)]).reshape(E_PAD // L, L)
    zeros_big = jnp.zeros((N_PAD, HID), f32)
    ones_deg = jnp.ones((L, DEGW), f32)

    degp = _degree(dst2d, zeros_big, ones_deg)
    d0 = degp[0, :N, :1]
    d1 = degp[1, :N, :1]

    A = bn_gamma / jnp.sqrt(bn_rv + 1e-5)
    B = (convb - bn_rm) * A + bn_beta

    x_p = jnp.pad(x, ((0, 0), (0, HID - x.shape[1])))
    Wp_p = jnp.pad(Wp, ((0, HID - Wp.shape[0]), (0, 0)))
    h, g = pl.pallas_call(
        _preprep_body,
        grid=(N // _BS,),
        in_specs=[pl.BlockSpec((_BS, HID), _node_spec),
                  pl.BlockSpec((HID, HID), _fixed_spec),
                  pl.BlockSpec((1, HID), _fixed_spec),
                  pl.BlockSpec((HID, HID), _fixed_spec),
                  pl.BlockSpec((_BS, 1), _node_spec),
                  pl.BlockSpec((_BS, 1), _node_spec)],
        out_specs=[pl.BlockSpec((_BS, HID), _node_spec),
                   pl.BlockSpec((_BS, HID), _node_spec)],
        out_shape=[jax.ShapeDtypeStruct((N, HID), f32),
                   jax.ShapeDtypeStruct((N, HID), f32)],
    )(x_p, Wp_p, bp[None, :], convW[0], d0, d1)

    postprep = pl.pallas_call(
        _postprep_body,
        grid=(N // _BS,),
        in_specs=[pl.BlockSpec((_BS, HID), _node_spec),
                  pl.BlockSpec((_BS, HID), _node_spec),
                  pl.BlockSpec((_BS, HID), _node_spec),
                  pl.BlockSpec((_BS, 1), _node_spec),
                  pl.BlockSpec((_BS, 1), _node_spec),
                  pl.BlockSpec((1, HID), _fixed_spec),
                  pl.BlockSpec((1, HID), _fixed_spec),
                  pl.BlockSpec((HID, HID), _fixed_spec)],
        out_specs=[pl.BlockSpec((_BS, HID), _node_spec),
                   pl.BlockSpec((_BS, HID), _node_spec)],
        out_shape=[jax.ShapeDtypeStruct((N, HID), f32),
                   jax.ShapeDtypeStruct((N, HID), f32)],
    )

    post = pl.pallas_call(
        _post_body,
        grid=(N // _BS,),
        in_specs=[pl.BlockSpec((_BS, HID), _node_spec),
                  pl.BlockSpec((_BS, HID), _node_spec),
                  pl.BlockSpec((_BS, HID), _node_spec),
                  pl.BlockSpec((_BS, 1), _node_spec),
                  pl.BlockSpec((_BS, 1), _node_spec),
                  pl.BlockSpec((1, HID), _fixed_spec),
                  pl.BlockSpec((1, HID), _fixed_spec)],
        out_specs=pl.BlockSpec((_BS, HID), _node_spec),
        out_shape=jax.ShapeDtypeStruct((N, HID), f32),
    )

    for i in range(LAYERS):
        accp = _edge_scatter(g, src2d, dst2d, zeros_big)
        if i < LAYERS - 1:
            h, g = postprep(accp[0, :N], accp[1, :N], g, d0, d1,
                            A[i][None, :], B[i][None, :], convW[i + 1])
        else:
            h = post(accp[0, :N], accp[1, :N], g, d0, d1,
                     A[i][None, :], B[i][None, :])

    node_features = h

    gsum, gmax, gcnt = pl.pallas_call(
        _pool_body,
        grid=(N // _BS,),
        in_specs=[pl.BlockSpec((_BS, HID), _node_spec),
                  pl.BlockSpec((1, 1, _BS), lambda i: (i, 0, 0)),
                  pl.BlockSpec((_BS, 1), _node_spec)],
        out_specs=[pl.BlockSpec((GRAPHS, HID), _fixed_spec),
                   pl.BlockSpec((GRAPHS, HID), _fixed_spec),
                   pl.BlockSpec((GRAPHS, 1), _fixed_spec)],
        out_shape=[jax.ShapeDtypeStruct((GRAPHS, HID), f32),
                   jax.ShapeDtypeStruct((GRAPHS, HID), f32),
                   jax.ShapeDtypeStruct((GRAPHS, 1), f32)],
    )(h, batch.reshape(N // _BS, 1, _BS), batch[:, None])

    mol = pl.pallas_call(
        _mlp_body,
        in_specs=[pl.BlockSpec((GRAPHS, HID), None),
                  pl.BlockSpec((GRAPHS, HID), None),
                  pl.BlockSpec((GRAPHS, 1), None),
                  pl.BlockSpec((HID, HID), None),
                  pl.BlockSpec((HID, HID), None),
                  pl.BlockSpec((1, HID), None),
                  pl.BlockSpec((HID, OUT), None),
                  pl.BlockSpec((1, OUT), None)],
        out_specs=pl.BlockSpec((GRAPHS, OUT), None),
        out_shape=jax.ShapeDtypeStruct((GRAPHS, OUT), f32),
    )(gsum, gmax, gcnt, Wo1[:HID], Wo1[HID:], bo1[None, :],
      Wo2, bo2[None, :])

    return (mol, node_features)
